# baseline (speedup 1.0000x reference)
import functools

import jax
import jax.numpy as jnp
from jax import lax
from jax.experimental import pallas as pl
from jax.experimental.pallas import tpu as pltpu
from jax.experimental.pallas import tpu_sc as plsc

_VOCAB = 1000000
_EMBED = 64
_BATCH = 4096
_SEQ = 200
_N = _BATCH * _SEQ
_NC = 2
_NS = 16
_NW = _NC * _NS
_PER_W = _N // _NW
_C = 512
_STEPS = _PER_W // _C

_mesh = plsc.VectorSubcoreMesh(core_axis_name="c", subcore_axis_name="s")


@functools.partial(
    pl.kernel,
    mesh=_mesh,
    compiler_params=pltpu.CompilerParams(use_tc_tiling_on_sc=False),
    out_type=jax.ShapeDtypeStruct((_N, _EMBED), jnp.float32),
    scratch_types=[
        pltpu.VMEM((_C,), jnp.int32),
        pltpu.VMEM((_C, _EMBED), jnp.float32),
        pltpu.SemaphoreType.DMA,
    ],
)
def _embed_lookup(idx_hbm, table_hbm, out_hbm, idx_v, rows_v, sem):
    wid = lax.axis_index("s") * _NC + lax.axis_index("c")
    base0 = wid * _PER_W

    def body(j, carry):
        base = pl.multiple_of(base0 + j * _C, 8)
        pltpu.sync_copy(idx_hbm.at[pl.ds(base, _C)], idx_v)
        pltpu.async_copy(table_hbm.at[idx_v], rows_v, sem).wait()
        pltpu.sync_copy(rows_v, out_hbm.at[pl.ds(base, _C)])
        return carry

    lax.fori_loop(0, _STEPS, body, 0)


def kernel(x, table):
    flat = x.reshape(_N)
    out = _embed_lookup(flat, table)
    return out.reshape(_BATCH, _SEQ, _EMBED)


# speedup vs baseline: 1.0438x; 1.0438x over previous
import functools

import jax
import jax.numpy as jnp
from jax import lax
from jax.experimental import pallas as pl
from jax.experimental.pallas import tpu as pltpu
from jax.experimental.pallas import tpu_sc as plsc

_VOCAB = 1000000
_EMBED = 64
_BATCH = 4096
_SEQ = 200
_N = _BATCH * _SEQ
_NC = 2
_NS = 16
_NW = _NC * _NS
_PER_W = _N // _NW
_C = 320
_NB = 4
_L = 2
_STEPS = _PER_W // _C
_GROUPS = _STEPS // _NB

_mesh = plsc.VectorSubcoreMesh(core_axis_name="c", subcore_axis_name="s")


@functools.partial(
    pl.kernel,
    mesh=_mesh,
    compiler_params=pltpu.CompilerParams(use_tc_tiling_on_sc=False),
    out_type=jax.ShapeDtypeStruct((_N, _EMBED), jnp.float32),
    scratch_types=[
        pltpu.VMEM((_PER_W,), jnp.int32),
        pltpu.VMEM((_NB, _C, _EMBED), jnp.float32),
        pltpu.SemaphoreType.DMA((_NB,)),
        pltpu.SemaphoreType.DMA((_NB,)),
    ],
)
def _embed_lookup(idx_hbm, table_hbm, out_hbm, idx_v, rows_v, gat_sem, out_sem):
    wid = lax.axis_index("s") * _NC + lax.axis_index("c")
    wbase = pl.multiple_of(wid * _PER_W, 8)
    pltpu.sync_copy(idx_hbm.at[pl.ds(wbase, _PER_W)], idx_v)

    def start_gather(b, c):
        pltpu.make_async_copy(
            table_hbm.at[idx_v.at[pl.ds(c * _C, _C)]],
            rows_v.at[b],
            gat_sem.at[b],
        ).start()

    def wait_gather(b):
        pltpu.make_async_copy(
            table_hbm.at[idx_v.at[pl.ds(0, _C)]],
            rows_v.at[b],
            gat_sem.at[b],
        ).wait()

    def start_out(b, c):
        dst = out_hbm.at[pl.ds(pl.multiple_of(wbase + c * _C, 8), _C)]
        pltpu.make_async_copy(rows_v.at[b], dst, out_sem.at[b]).start()

    def wait_out(b):
        dst = out_hbm.at[pl.ds(wbase, _C)]
        pltpu.make_async_copy(rows_v.at[b], dst, out_sem.at[b]).wait()

    def body(g, carry):
        for b in range(_NB):
            c = g * _NB + b
            @pl.when(g >= 1)
            def _():
                wait_out(b)

            start_gather(b, c)

            b2 = (b - _L) % _NB
            c2 = c - _L

            @pl.when(c2 >= 0)
            def _():
                wait_gather(b2)
                start_out(b2, c2)

        return carry

    lax.fori_loop(0, _GROUPS, body, 0)

    for k in range(_L):
        c2 = _STEPS - _L + k
        b2 = c2 % _NB
        wait_gather(b2)
        start_out(b2, c2)
    for b in range(_NB):
        wait_out(b)


def kernel(x, table):
    flat = x.reshape(_N)
    out = _embed_lookup(flat, table)
    return out.reshape(_BATCH, _SEQ, _EMBED)


# speedup vs baseline: 1.0458x; 1.0019x over previous
import functools

import jax
import jax.numpy as jnp
from jax import lax
from jax.experimental import pallas as pl
from jax.experimental.pallas import tpu as pltpu
from jax.experimental.pallas import tpu_sc as plsc

_VOCAB = 1000000
_EMBED = 64
_BATCH = 4096
_SEQ = 200
_NC = 2
_NS = 16
_NW = _NC * _NS
_ROWS_W = _BATCH // _NW
_NB = 4
_L = 2
_GROUPS = _ROWS_W // _NB

_mesh = plsc.VectorSubcoreMesh(core_axis_name="c", subcore_axis_name="s")


@functools.partial(
    pl.kernel,
    mesh=_mesh,
    compiler_params=pltpu.CompilerParams(use_tc_tiling_on_sc=False),
    out_type=jax.ShapeDtypeStruct((_BATCH, _SEQ, _EMBED), jnp.float32),
    scratch_types=[
        pltpu.VMEM((_ROWS_W, _SEQ), jnp.int32),
        pltpu.VMEM((_NB, _SEQ, _EMBED), jnp.float32),
        pltpu.SemaphoreType.DMA((_NB,)),
        pltpu.SemaphoreType.DMA((_NB,)),
    ],
)
def _embed_lookup(x_hbm, table_hbm, out_hbm, idx_v, rows_v, gat_sem, out_sem):
    wid = lax.axis_index("s") * _NC + lax.axis_index("c")
    wrow = pl.multiple_of(wid * _ROWS_W, 8)
    pltpu.sync_copy(x_hbm.at[pl.ds(wrow, _ROWS_W)], idx_v)

    def start_gather(b, r):
        pltpu.make_async_copy(
            table_hbm.at[idx_v.at[r]], rows_v.at[b], gat_sem.at[b]
        ).start()

    def wait_gather(b):
        pltpu.make_async_copy(
            table_hbm.at[idx_v.at[0]], rows_v.at[b], gat_sem.at[b]
        ).wait()

    def start_out(b, r):
        pltpu.make_async_copy(
            rows_v.at[b], out_hbm.at[wrow + r], out_sem.at[b]
        ).start()

    def wait_out(b):
        pltpu.make_async_copy(
            rows_v.at[b], out_hbm.at[wrow], out_sem.at[b]
        ).wait()

    def body(g, carry):
        for b in range(_NB):
            r = g * _NB + b
            @pl.when(g >= 1)
            def _():
                wait_out(b)

            start_gather(b, r)

            b2 = (b - _L) % _NB
            r2 = r - _L

            @pl.when(r2 >= 0)
            def _():
                wait_gather(b2)
                start_out(b2, r2)

        return carry

    lax.fori_loop(0, _GROUPS, body, 0)

    for k in range(_L):
        r2 = _ROWS_W - _L + k
        b2 = r2 % _NB
        wait_gather(b2)
        start_out(b2, r2)
    for b in range(_NB):
        wait_out(b)


def kernel(x, table):
    return _embed_lookup(x, table)


# speedup vs baseline: 1.3889x; 1.3280x over previous
import functools

import jax
import jax.numpy as jnp
from jax import lax
from jax.experimental import pallas as pl
from jax.experimental.pallas import tpu as pltpu
from jax.experimental.pallas import tpu_sc as plsc

_VOCAB = 1000000
_EMBED = 64
_BATCH = 4096
_SEQ = 200
_NC = 2
_NS = 16
_NW = _NC * _NS
_ROWS_W = _BATCH // _NW
_NB = 4
_L = 2
_GROUPS = _ROWS_W // _NB

_mesh = plsc.VectorSubcoreMesh(core_axis_name="c", subcore_axis_name="s")


@functools.partial(
    pl.kernel,
    mesh=_mesh,
    compiler_params=pltpu.CompilerParams(use_tc_tiling_on_sc=False),
    out_type=jax.ShapeDtypeStruct((_BATCH, _SEQ, 2 * _EMBED), jnp.float32),
    scratch_types=[
        pltpu.VMEM((_ROWS_W, _SEQ), jnp.int32),
        pltpu.VMEM((_NB, _SEQ, _EMBED), jnp.float32),
        pltpu.SemaphoreType.DMA((_NB,)),
        pltpu.SemaphoreType.DMA((_NB,)),
    ],
)
def _embed_lookup(x_hbm, table_hbm, out_hbm, idx_v, rows_v, gat_sem, out_sem):
    wid = lax.axis_index("s") * _NC + lax.axis_index("c")
    wrow = pl.multiple_of(wid * _ROWS_W, 8)
    pltpu.sync_copy(x_hbm.at[pl.ds(wrow, _ROWS_W)], idx_v)

    def start_gather(b, r):
        pltpu.make_async_copy(
            table_hbm.at[idx_v.at[r]], rows_v.at[b], gat_sem.at[b]
        ).start()

    def wait_gather(b):
        pltpu.make_async_copy(
            table_hbm.at[idx_v.at[0]], rows_v.at[b], gat_sem.at[b]
        ).wait()

    def start_out(b, r):
        pltpu.make_async_copy(
            rows_v.at[b],
            out_hbm.at[wrow + r, :, pl.ds(0, _EMBED)],
            out_sem.at[b],
        ).start()

    def wait_out(b):
        pltpu.make_async_copy(
            rows_v.at[b],
            out_hbm.at[wrow, :, pl.ds(0, _EMBED)],
            out_sem.at[b],
        ).wait()

    def body(g, carry):
        for b in range(_NB):
            r = g * _NB + b
            @pl.when(g >= 1)
            def _():
                wait_out(b)

            start_gather(b, r)

            b2 = (b - _L) % _NB
            r2 = r - _L

            @pl.when(r2 >= 0)
            def _():
                wait_gather(b2)
                start_out(b2, r2)

        return carry

    lax.fori_loop(0, _GROUPS, body, 0)

    for k in range(_L):
        r2 = _ROWS_W - _L + k
        b2 = r2 % _NB
        wait_gather(b2)
        start_out(b2, r2)
    for b in range(_NB):
        wait_out(b)


def kernel(x, table):
    return _embed_lookup(x, table)[..., :_EMBED]
